# baseline (speedup 1.0000x reference)
import jax
import jax.numpy as jnp
from jax.experimental import pallas as pl
from jax.experimental.pallas import tpu as pltpu

TB = 128
NCHUNK = 4


def _body(x_ref, srow_ref, rrow_ref, scol_ref, rcol_ref,
          srep_ref, rrep_ref, arep_ref, brep_ref, crep_ref,
          irow_ref, w1f_ref, b1f_ref, w2c_ref, b2_ref, cw_ref, cb2_ref,
          out_ref, a_scr, e_scr, r_scr, sm_scr):
    f32 = jnp.float32
    tb = x_ref.shape[0]
    n = x_ref.shape[1]
    p = srow_ref.shape[1]
    hdim = cw_ref.shape[1]
    ka = a_scr.shape[0]

    @pl.when(pl.program_id(0) == 0)
    def _init():
        ja = jax.lax.broadcasted_iota(jnp.int32, (ka, hdim * p), 0)
        a_scr[...] = (jnp.where(ja == srep_ref[...], arep_ref[...], 0.0)
                      + jnp.where(ja == rrep_ref[...], brep_ref[...], 0.0)
                      + jnp.where(ja == n, crep_ref[...], 0.0))
        je = jax.lax.broadcasted_iota(jnp.int32, (ka, hdim * n), 0)
        e_scr[...] = (jnp.where(je == irow_ref[...], w1f_ref[...], 0.0)
                      + jnp.where(je == n, b1f_ref[...], 0.0))
        k_iota = jax.lax.broadcasted_iota(jnp.int32, (hdim * n, n), 0)
        i_iota = jax.lax.broadcasted_iota(jnp.int32, (hdim * n, n), 1)
        r_scr[...] = ((k_iota % n) == i_iota).astype(f32) * w2c_ref[...]
        iota_pn = jax.lax.broadcasted_iota(jnp.int32, (p, n), 1)
        sm_scr[...] = ((iota_pn == scol_ref[...]).astype(f32)
                       - (iota_pn == rcol_ref[...]).astype(f32))

    x = x_ref[...]
    xa = jnp.concatenate(
        [x, jnp.ones((tb, 1), f32), jnp.zeros((tb, ka - n - 1), f32)],
        axis=1)

    pre_i = jnp.dot(xa, e_scr[...], preferred_element_type=f32)
    ti = jnp.tanh(pre_i)
    intr = jnp.dot(ti, r_scr[...], preferred_element_type=f32) + b2_ref[...]

    contrib = jnp.full((tb, p), cb2_ref[0, 0], dtype=f32)
    hc = hdim // NCHUNK
    for k in range(NCHUNK):
        pre_k = jnp.dot(xa, a_scr[:, k * hc * p:(k + 1) * hc * p],
                        preferred_element_type=f32)
        for j in range(hc):
            h = k * hc + j
            t = jnp.tanh(pre_k[:, j * p:(j + 1) * p])
            contrib = contrib + cw_ref[3, h] * t

    coup = jnp.dot(contrib, sm_scr[...], preferred_element_type=f32)
    out_ref[...] = intr + coup


def kernel(x, pins, W1, b1, W2, b2, cW1, cb1, cW2, cb2):
    B, N = x.shape
    P = pins.shape[0]
    H = cW1.shape[1]
    f32 = jnp.float32
    KA = N + 8
    srow = pins[:, 0].reshape(1, P)
    rrow = pins[:, 1].reshape(1, P)
    scol = pins[:, 0:1]
    rcol = pins[:, 1:2]
    srep = jnp.tile(pins[:, 0], H).reshape(1, H * P)
    rrep = jnp.tile(pins[:, 1], H).reshape(1, H * P)
    arep = jnp.repeat(cW1[0, :], P).reshape(1, H * P)
    brep = jnp.repeat(cW1[1, :], P).reshape(1, H * P)
    crep = jnp.repeat(cb1, P).reshape(1, H * P)
    irow = jnp.tile(jnp.arange(N, dtype=jnp.int32), H).reshape(1, H * N)
    w1f = W1.reshape(N, H).T.reshape(1, H * N)
    b1f = b1.T.reshape(1, H * N)
    w2c = W2.reshape(N, H).T.reshape(H * N, 1)
    b2r = b2.reshape(1, N)
    cw = jnp.concatenate([cW1[0:1, :], cW1[1:2, :],
                          cb1.reshape(1, H), cW2.reshape(1, H)], axis=0)
    cb2r = cb2.reshape(1, 1)

    full = lambda shape: pl.BlockSpec(shape, lambda i: (0,) * len(shape))
    return pl.pallas_call(
        _body,
        grid=(B // TB,),
        in_specs=[
            pl.BlockSpec((TB, N), lambda i: (i, 0)),
            full((1, P)), full((1, P)), full((P, 1)), full((P, 1)),
            full((1, H * P)), full((1, H * P)), full((1, H * P)),
            full((1, H * P)), full((1, H * P)),
            full((1, H * N)), full((1, H * N)), full((1, H * N)),
            full((H * N, 1)), full((1, N)),
            pl.BlockSpec(memory_space=pltpu.SMEM),
            pl.BlockSpec(memory_space=pltpu.SMEM),
        ],
        out_specs=pl.BlockSpec((TB, N), lambda i: (i, 0)),
        out_shape=jax.ShapeDtypeStruct((B, N), x.dtype),
        scratch_shapes=[
            pltpu.VMEM((KA, H * P), f32),
            pltpu.VMEM((KA, H * N), f32),
            pltpu.VMEM((H * N, N), f32),
            pltpu.VMEM((P, N), f32),
        ],
    )(x, srow, rrow, scol, rcol, srep, rrep, arep, brep, crep,
      irow, w1f, b1f, w2c, b2r, cw, cb2r)


# speedup vs baseline: 1.0852x; 1.0852x over previous
import jax
import jax.numpy as jnp
from jax.experimental import pallas as pl
from jax.experimental.pallas import tpu as pltpu

TB = 256
NCHUNK = 4


def _body(x_ref, srow_ref, rrow_ref, scol_ref, rcol_ref,
          srep_ref, rrep_ref, arep_ref, brep_ref, crep_ref,
          irow_ref, w1f_ref, b1f_ref, w2f_ref, b2_ref, cw_ref, cb2_ref,
          out_ref, a_scr, e_scr, r_scr, sm_scr):
    f32 = jnp.float32
    tb = x_ref.shape[0]
    n = x_ref.shape[1]
    p = srow_ref.shape[1]
    hdim = cw_ref.shape[1]
    ka = a_scr.shape[0]

    @pl.when(pl.program_id(0) == 0)
    def _init():
        ja = jax.lax.broadcasted_iota(jnp.int32, (ka, hdim * p), 0)
        a_scr[...] = (jnp.where(ja == srep_ref[...], arep_ref[...], 0.0)
                      + jnp.where(ja == rrep_ref[...], brep_ref[...], 0.0)
                      + jnp.where(ja == n, crep_ref[...], 0.0))
        je = jax.lax.broadcasted_iota(jnp.int32, (ka, hdim * n), 0)
        e_scr[...] = (jnp.where(je == irow_ref[...], w1f_ref[...], 0.0)
                      + jnp.where(je == n, b1f_ref[...], 0.0))
        k_iota = jax.lax.broadcasted_iota(jnp.int32, (hdim * n, n), 0)
        i_iota = jax.lax.broadcasted_iota(jnp.int32, (hdim * n, n), 1)
        r_scr[...] = ((k_iota % n) == i_iota).astype(f32)
        iota_pn = jax.lax.broadcasted_iota(jnp.int32, (p, n), 1)
        sm_scr[...] = ((iota_pn == scol_ref[...]).astype(f32)
                       - (iota_pn == rcol_ref[...]).astype(f32))

    x = x_ref[...]
    xa = jnp.concatenate(
        [x, jnp.ones((tb, 1), f32), jnp.zeros((tb, ka - n - 1), f32)],
        axis=1)

    pre_i = jnp.dot(xa, e_scr[...], preferred_element_type=f32)
    ti = jnp.tanh(pre_i) * w2f_ref[...]
    intr = jnp.dot(ti, r_scr[...], preferred_element_type=f32) + b2_ref[...]

    contrib = jnp.full((tb, p), cb2_ref[0, 0], dtype=f32)
    hc = hdim // NCHUNK
    for k in range(NCHUNK):
        pre_k = jnp.dot(xa, a_scr[:, k * hc * p:(k + 1) * hc * p],
                        preferred_element_type=f32)
        for j in range(hc):
            h = k * hc + j
            t = jnp.tanh(pre_k[:, j * p:(j + 1) * p])
            contrib = contrib + cw_ref[3, h] * t

    coup = jnp.dot(contrib, sm_scr[...], preferred_element_type=f32)
    out_ref[...] = intr + coup


def kernel(x, pins, W1, b1, W2, b2, cW1, cb1, cW2, cb2):
    B, N = x.shape
    P = pins.shape[0]
    H = cW1.shape[1]
    f32 = jnp.float32
    KA = N + 8
    srow = pins[:, 0].reshape(1, P)
    rrow = pins[:, 1].reshape(1, P)
    scol = pins[:, 0:1]
    rcol = pins[:, 1:2]
    srep = jnp.tile(pins[:, 0], H).reshape(1, H * P)
    rrep = jnp.tile(pins[:, 1], H).reshape(1, H * P)
    arep = jnp.repeat(cW1[0, :], P).reshape(1, H * P)
    brep = jnp.repeat(cW1[1, :], P).reshape(1, H * P)
    crep = jnp.repeat(cb1, P).reshape(1, H * P)
    irow = jnp.tile(jnp.arange(N, dtype=jnp.int32), H).reshape(1, H * N)
    w1f = W1.reshape(N, H).T.reshape(1, H * N)
    b1f = b1.T.reshape(1, H * N)
    w2f = W2.reshape(N, H).T.reshape(1, H * N)
    b2r = b2.reshape(1, N)
    cw = jnp.concatenate([cW1[0:1, :], cW1[1:2, :],
                          cb1.reshape(1, H), cW2.reshape(1, H)], axis=0)
    cb2r = cb2.reshape(1, 1)

    full = lambda shape: pl.BlockSpec(shape, lambda i: (0,) * len(shape))
    return pl.pallas_call(
        _body,
        grid=(B // TB,),
        in_specs=[
            pl.BlockSpec((TB, N), lambda i: (i, 0)),
            full((1, P)), full((1, P)), full((P, 1)), full((P, 1)),
            full((1, H * P)), full((1, H * P)), full((1, H * P)),
            full((1, H * P)), full((1, H * P)),
            full((1, H * N)), full((1, H * N)), full((1, H * N)),
            full((1, H * N)), full((1, N)),
            pl.BlockSpec(memory_space=pltpu.SMEM),
            pl.BlockSpec(memory_space=pltpu.SMEM),
        ],
        out_specs=pl.BlockSpec((TB, N), lambda i: (i, 0)),
        out_shape=jax.ShapeDtypeStruct((B, N), x.dtype),
        scratch_shapes=[
            pltpu.VMEM((KA, H * P), f32),
            pltpu.VMEM((KA, H * N), f32),
            pltpu.VMEM((H * N, N), f32),
            pltpu.VMEM((P, N), f32),
        ],
    )(x, srow, rrow, scol, rcol, srep, rrep, arep, brep, crep,
      irow, w1f, b1f, w2f, b2r, cw, cb2r)


# speedup vs baseline: 1.1085x; 1.0215x over previous
import jax
import jax.numpy as jnp
from jax.experimental import pallas as pl
from jax.experimental.pallas import tpu as pltpu

TB = 512
NCHUNK = 4


def _body(x_ref, srow_ref, rrow_ref, scol_ref, rcol_ref,
          srep_ref, rrep_ref, arep_ref, brep_ref, crep_ref,
          irow_ref, w1f_ref, b1f_ref, w2f_ref, b2_ref, cw_ref, cb2_ref,
          out_ref, a_scr, e_scr, r_scr, sm_scr):
    f32 = jnp.float32
    tb = x_ref.shape[0]
    n = x_ref.shape[1]
    p = srow_ref.shape[1]
    hdim = cw_ref.shape[1]
    ka = a_scr.shape[0]

    @pl.when(pl.program_id(0) == 0)
    def _init():
        ja = jax.lax.broadcasted_iota(jnp.int32, (ka, hdim * p), 0)
        a_scr[...] = (jnp.where(ja == srep_ref[...], arep_ref[...], 0.0)
                      + jnp.where(ja == rrep_ref[...], brep_ref[...], 0.0)
                      + jnp.where(ja == n, crep_ref[...], 0.0))
        je = jax.lax.broadcasted_iota(jnp.int32, (ka, hdim * n), 0)
        e_scr[...] = (jnp.where(je == irow_ref[...], w1f_ref[...], 0.0)
                      + jnp.where(je == n, b1f_ref[...], 0.0))
        k_iota = jax.lax.broadcasted_iota(jnp.int32, (hdim * n, n), 0)
        i_iota = jax.lax.broadcasted_iota(jnp.int32, (hdim * n, n), 1)
        r_scr[...] = ((k_iota % n) == i_iota).astype(f32)
        iota_pn = jax.lax.broadcasted_iota(jnp.int32, (p, n), 1)
        sm_scr[...] = ((iota_pn == scol_ref[...]).astype(f32)
                       - (iota_pn == rcol_ref[...]).astype(f32))

    x = x_ref[...]
    xa = jnp.concatenate(
        [x, jnp.ones((tb, 1), f32), jnp.zeros((tb, ka - n - 1), f32)],
        axis=1)

    pre_i = jnp.dot(xa, e_scr[...], preferred_element_type=f32)
    ti = jnp.tanh(pre_i) * w2f_ref[...]
    intr = jnp.dot(ti, r_scr[...], preferred_element_type=f32) + b2_ref[...]

    contrib = jnp.full((tb, p), cb2_ref[0, 0], dtype=f32)
    hc = hdim // NCHUNK
    for k in range(NCHUNK):
        pre_k = jnp.dot(xa, a_scr[:, k * hc * p:(k + 1) * hc * p],
                        preferred_element_type=f32)
        for j in range(hc):
            h = k * hc + j
            t = jnp.tanh(pre_k[:, j * p:(j + 1) * p])
            contrib = contrib + cw_ref[3, h] * t

    coup = jnp.dot(contrib, sm_scr[...], preferred_element_type=f32)
    out_ref[...] = intr + coup


def kernel(x, pins, W1, b1, W2, b2, cW1, cb1, cW2, cb2):
    B, N = x.shape
    P = pins.shape[0]
    H = cW1.shape[1]
    f32 = jnp.float32
    KA = N + 8
    srow = pins[:, 0].reshape(1, P)
    rrow = pins[:, 1].reshape(1, P)
    scol = pins[:, 0:1]
    rcol = pins[:, 1:2]
    srep = jnp.tile(pins[:, 0], H).reshape(1, H * P)
    rrep = jnp.tile(pins[:, 1], H).reshape(1, H * P)
    arep = jnp.repeat(cW1[0, :], P).reshape(1, H * P)
    brep = jnp.repeat(cW1[1, :], P).reshape(1, H * P)
    crep = jnp.repeat(cb1, P).reshape(1, H * P)
    irow = jnp.tile(jnp.arange(N, dtype=jnp.int32), H).reshape(1, H * N)
    w1f = W1.reshape(N, H).T.reshape(1, H * N)
    b1f = b1.T.reshape(1, H * N)
    w2f = W2.reshape(N, H).T.reshape(1, H * N)
    b2r = b2.reshape(1, N)
    cw = jnp.concatenate([cW1[0:1, :], cW1[1:2, :],
                          cb1.reshape(1, H), cW2.reshape(1, H)], axis=0)
    cb2r = cb2.reshape(1, 1)

    full = lambda shape: pl.BlockSpec(shape, lambda i: (0,) * len(shape))
    return pl.pallas_call(
        _body,
        grid=(B // TB,),
        in_specs=[
            pl.BlockSpec((TB, N), lambda i: (i, 0)),
            full((1, P)), full((1, P)), full((P, 1)), full((P, 1)),
            full((1, H * P)), full((1, H * P)), full((1, H * P)),
            full((1, H * P)), full((1, H * P)),
            full((1, H * N)), full((1, H * N)), full((1, H * N)),
            full((1, H * N)), full((1, N)),
            pl.BlockSpec(memory_space=pltpu.SMEM),
            pl.BlockSpec(memory_space=pltpu.SMEM),
        ],
        out_specs=pl.BlockSpec((TB, N), lambda i: (i, 0)),
        out_shape=jax.ShapeDtypeStruct((B, N), x.dtype),
        scratch_shapes=[
            pltpu.VMEM((KA, H * P), f32),
            pltpu.VMEM((KA, H * N), f32),
            pltpu.VMEM((H * N, N), f32),
            pltpu.VMEM((P, N), f32),
        ],
    )(x, srow, rrow, scol, rcol, srep, rrep, arep, brep, crep,
      irow, w1f, b1f, w2f, b2r, cw, cb2r)
